# Initial kernel scaffold; baseline (speedup 1.0000x reference)
import jax
import jax.numpy as jnp
from jax.experimental import pallas as pl


def kernel(rois, cls_prob, bbox_pred_tensor, im_info, gt_boxes):
    raise NotImplementedError("write your pallas kernel here")


# speedup vs baseline: 1.1692x; 1.1692x over previous
import functools

import jax
import jax.numpy as jnp
from jax import lax
from jax.experimental import pallas as pl
from jax.experimental.pallas import tpu as pltpu
from jax.experimental.pallas import tpu_sc as plsc

N = 5000
C = 81
G = 64
L = 16
FG_THRESH = 0.5


def _body(rois_hbm, cls_hbm, bpred_hbm, imb_hbm, gtaux_hbm, gcls_hbm,
          blob_hbm, ocls_hbm,
          cls_v, rois_v, imb_v, gtaux_v, gcls_v, idx_v, del_v, blob_v,
          ocls_v, sem, *, nc, b):
    ngrp = b // L
    wid = lax.axis_index("s") * nc + lax.axis_index("c")
    base = jnp.minimum(wid * b, N - b)

    pltpu.sync_copy(cls_hbm.at[pl.ds(base * C, b * C)], cls_v)
    pltpu.sync_copy(rois_hbm.at[pl.ds(base * 5, b * 5)], rois_v)
    pltpu.sync_copy(imb_hbm, imb_v)
    pltpu.sync_copy(gtaux_hbm, gtaux_v)
    pltpu.sync_copy(gcls_hbm, gcls_v)

    lanes = jnp.arange(L, dtype=jnp.int32)

    def pass1(g, carry):
        rows = g * L + lanes
        rbase = rows * C
        m = plsc.load_gather(cls_v, [rbase + 1])
        am = jnp.zeros((L,), jnp.int32)
        for cc in range(2, C):
            v = plsc.load_gather(cls_v, [rbase + cc])
            better = v > m
            am = jnp.where(better, cc - 1, am)
            m = jnp.where(better, v, m)
        fidx = ((base + rows) * C + am) * 4
        for j in range(4):
            idx_v[pl.ds(j * b + g * L, L)] = fidx + j
        return carry

    lax.fori_loop(0, ngrp, pass1, None, unroll=False)

    half = b // 2
    copies = []
    for j in range(4):
        for h in range(2):
            o = j * b + h * half
            copies.append(pltpu.async_copy(
                bpred_hbm.at[idx_v.at[pl.ds(o, half)]],
                del_v.at[pl.ds(o, half)], sem))
    for cp in copies:
        cp.wait()

    wlim = imb_v[pl.ds(0, L)]
    hlim = imb_v[pl.ds(L, L)]

    def pass2(g, carry):
        rows = g * L + lanes
        r5 = rows * 5
        x1 = plsc.load_gather(rois_v, [r5 + 1])
        y1 = plsc.load_gather(rois_v, [r5 + 2])
        x2 = plsc.load_gather(rois_v, [r5 + 3])
        y2 = plsc.load_gather(rois_v, [r5 + 4])
        dx = del_v[pl.ds(0 * b + g * L, L)] * 0.1
        dy = del_v[pl.ds(1 * b + g * L, L)] * 0.1
        dw = del_v[pl.ds(2 * b + g * L, L)] * 0.2
        dh = del_v[pl.ds(3 * b + g * L, L)] * 0.2
        w = x2 - x1 + 1.0
        h = y2 - y1 + 1.0
        cx = x1 + 0.5 * w
        cy = y1 + 0.5 * h
        pcx = dx * w + cx
        pcy = dy * h + cy
        pw = jnp.exp(dw) * w
        ph = jnp.exp(dh) * h
        bx1 = pcx - 0.5 * pw
        by1 = pcy - 0.5 * ph
        bx2 = pcx + 0.5 * pw
        by2 = pcy + 0.5 * ph
        zero = jnp.zeros((L,), jnp.float32)
        cx1 = jnp.minimum(jnp.maximum(bx1, zero), wlim)
        cy1 = jnp.minimum(jnp.maximum(by1, zero), hlim)
        cx2 = jnp.minimum(jnp.maximum(bx2, zero), wlim)
        cy2 = jnp.minimum(jnp.maximum(by2, zero), hlim)
        area_b = (cx2 - cx1 + 1.0) * (cy2 - cy1 + 1.0)

        m_iou = None
        am = jnp.zeros((L,), jnp.int32)
        for g2 in range(G):
            qx1 = gtaux_v[pl.ds((0 * G + g2) * L, L)]
            qy1 = gtaux_v[pl.ds((1 * G + g2) * L, L)]
            qx2 = gtaux_v[pl.ds((2 * G + g2) * L, L)]
            qy2 = gtaux_v[pl.ds((3 * G + g2) * L, L)]
            qa = gtaux_v[pl.ds((4 * G + g2) * L, L)]
            iw = jnp.maximum(
                jnp.minimum(cx2, qx2) - jnp.maximum(cx1, qx1) + 1.0, zero)
            ih = jnp.maximum(
                jnp.minimum(cy2, qy2) - jnp.maximum(cy1, qy1) + 1.0, zero)
            inter = iw * ih
            iou = inter / (area_b + qa - inter)
            if m_iou is None:
                m_iou = iou
            else:
                better = iou > m_iou
                am = jnp.where(better, g2, am)
                m_iou = jnp.where(better, iou, m_iou)

        tgt = plsc.load_gather(gcls_v, [am * L + lanes])
        fin = jnp.where(m_iou >= FG_THRESH, tgt, jnp.zeros((L,), jnp.int32))
        ocls_v[pl.ds(g * L, L)] = fin
        plsc.store_scatter(blob_v, [r5], zero)
        plsc.store_scatter(blob_v, [r5 + 1], cx1)
        plsc.store_scatter(blob_v, [r5 + 2], cy1)
        plsc.store_scatter(blob_v, [r5 + 3], cx2)
        plsc.store_scatter(blob_v, [r5 + 4], cy2)
        return carry

    lax.fori_loop(0, ngrp, pass2, None, unroll=False)

    pltpu.sync_copy(blob_v, blob_hbm.at[pl.ds(base * 5, b * 5)])
    pltpu.sync_copy(ocls_v, ocls_hbm.at[pl.ds(base, b)])


@jax.jit
def _run(rois_f, cls_f, bpred_f, imb, gtaux_f, gcls_f):
    info = plsc.get_sparse_core_info()
    nc, ns = info.num_cores, info.num_subcores
    nw = nc * ns
    b = -(-N // (nw * L)) * L
    mesh = plsc.VectorSubcoreMesh(core_axis_name="c", subcore_axis_name="s")
    kfn = pl.kernel(
        functools.partial(_body, nc=nc, b=b),
        out_type=[
            jax.ShapeDtypeStruct((N * 5,), jnp.float32),
            jax.ShapeDtypeStruct((N,), jnp.int32),
        ],
        mesh=mesh,
        compiler_params=pltpu.CompilerParams(needs_layout_passes=False),
        scratch_types=[
            pltpu.VMEM((b * C,), jnp.float32),
            pltpu.VMEM((b * 5,), jnp.float32),
            pltpu.VMEM((2 * L,), jnp.float32),
            pltpu.VMEM((5 * G * L,), jnp.float32),
            pltpu.VMEM((G * L,), jnp.int32),
            pltpu.VMEM((4 * b,), jnp.int32),
            pltpu.VMEM((4 * b,), jnp.float32),
            pltpu.VMEM((b * 5,), jnp.float32),
            pltpu.VMEM((b,), jnp.int32),
            pltpu.SemaphoreType.DMA,
        ],
    )
    return kfn(rois_f, cls_f, bpred_f, imb, gtaux_f, gcls_f)


def kernel(rois, cls_prob, bbox_pred_tensor, im_info, gt_boxes):
    gt_valid = jnp.cumsum((gt_boxes[:, 2] < 0.01).astype(jnp.int32)) == 0
    qx1 = jnp.where(gt_valid, gt_boxes[:, 0], 2e9)
    qy1 = jnp.where(gt_valid, gt_boxes[:, 1], 2e9)
    qx2 = jnp.where(gt_valid, gt_boxes[:, 2], 0.0)
    qy2 = jnp.where(gt_valid, gt_boxes[:, 3], 0.0)
    qa = (qx2 - qx1 + 1.0) * (qy2 - qy1 + 1.0)
    gtaux = jnp.broadcast_to(
        jnp.stack([qx1, qy1, qx2, qy2, qa])[:, :, None], (5, G, L))
    gcls = jnp.broadcast_to(
        gt_boxes[:, 4].astype(jnp.int32)[:, None], (G, L))
    imb = jnp.concatenate([
        jnp.full((L,), im_info[0, 1] - 1.0, jnp.float32),
        jnp.full((L,), im_info[0, 0] - 1.0, jnp.float32),
    ])
    blob_f, ocls = _run(
        rois.reshape(-1), cls_prob.reshape(-1), bbox_pred_tensor.reshape(-1),
        imb, gtaux.reshape(-1), gcls.reshape(-1))
    return blob_f.reshape(N, 5), ocls


# speedup vs baseline: 1.8073x; 1.5458x over previous
import functools

import jax
import jax.numpy as jnp
from jax import lax
from jax.experimental import pallas as pl
from jax.experimental.pallas import tpu as pltpu
from jax.experimental.pallas import tpu_sc as plsc

N = 5000
C = 81
G = 64
L = 16
FG_THRESH = 0.5


def _body(rois_hbm, cls_hbm, bpred_hbm, imb_hbm, gtaux_hbm, gcls_hbm,
          blob_hbm, ocls_hbm,
          cls_v, rois_v, bpred_v, imb_v, gtaux_v, gcls_v, blob_v,
          ocls_v, *, nc, b):
    ngrp = b // L
    wid = lax.axis_index("s") * nc + lax.axis_index("c")
    base = jnp.minimum(wid * b, N - b)

    pltpu.sync_copy(cls_hbm.at[pl.ds(base, b)], cls_v)
    pltpu.sync_copy(rois_hbm.at[pl.ds(base, b)], rois_v)
    pltpu.sync_copy(bpred_hbm.at[pl.ds(base, b)], bpred_v)
    pltpu.sync_copy(imb_hbm, imb_v)
    pltpu.sync_copy(gtaux_hbm, gtaux_v)
    pltpu.sync_copy(gcls_hbm, gcls_v)

    lanes = jnp.arange(L, dtype=jnp.int32)
    wlim = imb_v[pl.ds(0, L)]
    hlim = imb_v[pl.ds(L, L)]

    def fc(k):
        return jnp.full((L,), k, jnp.int32)

    def grp(g, carry):
        rows = g * L + lanes
        m = plsc.load_gather(cls_v, [rows, fc(1)])
        am = jnp.zeros((L,), jnp.int32)
        for cc in range(2, C):
            v = plsc.load_gather(cls_v, [rows, fc(cc)])
            better = v > m
            am = jnp.where(better, cc - 1, am)
            m = jnp.where(better, v, m)
        am4 = am * 4
        dx = plsc.load_gather(bpred_v, [rows, am4]) * 0.1
        dy = plsc.load_gather(bpred_v, [rows, am4 + 1]) * 0.1
        dw = plsc.load_gather(bpred_v, [rows, am4 + 2]) * 0.2
        dh = plsc.load_gather(bpred_v, [rows, am4 + 3]) * 0.2
        x1 = plsc.load_gather(rois_v, [rows, fc(1)])
        y1 = plsc.load_gather(rois_v, [rows, fc(2)])
        x2 = plsc.load_gather(rois_v, [rows, fc(3)])
        y2 = plsc.load_gather(rois_v, [rows, fc(4)])
        w = x2 - x1 + 1.0
        h = y2 - y1 + 1.0
        cx = x1 + 0.5 * w
        cy = y1 + 0.5 * h
        pcx = dx * w + cx
        pcy = dy * h + cy
        pw = jnp.exp(dw) * w
        ph = jnp.exp(dh) * h
        bx1 = pcx - 0.5 * pw
        by1 = pcy - 0.5 * ph
        bx2 = pcx + 0.5 * pw
        by2 = pcy + 0.5 * ph
        zero = jnp.zeros((L,), jnp.float32)
        cx1 = jnp.minimum(jnp.maximum(bx1, zero), wlim)
        cy1 = jnp.minimum(jnp.maximum(by1, zero), hlim)
        cx2 = jnp.minimum(jnp.maximum(bx2, zero), wlim)
        cy2 = jnp.minimum(jnp.maximum(by2, zero), hlim)
        area_b = (cx2 - cx1 + 1.0) * (cy2 - cy1 + 1.0)

        m_iou = None
        am2 = jnp.zeros((L,), jnp.int32)
        for g2 in range(G):
            qx1 = gtaux_v[pl.ds((0 * G + g2) * L, L)]
            qy1 = gtaux_v[pl.ds((1 * G + g2) * L, L)]
            qx2 = gtaux_v[pl.ds((2 * G + g2) * L, L)]
            qy2 = gtaux_v[pl.ds((3 * G + g2) * L, L)]
            qa = gtaux_v[pl.ds((4 * G + g2) * L, L)]
            iw = jnp.maximum(
                jnp.minimum(cx2, qx2) - jnp.maximum(cx1, qx1) + 1.0, zero)
            ih = jnp.maximum(
                jnp.minimum(cy2, qy2) - jnp.maximum(cy1, qy1) + 1.0, zero)
            inter = iw * ih
            iou = inter / (area_b + qa - inter)
            if m_iou is None:
                m_iou = iou
            else:
                better = iou > m_iou
                am2 = jnp.where(better, g2, am2)
                m_iou = jnp.where(better, iou, m_iou)

        tgt = plsc.load_gather(gcls_v, [am2 * L + lanes])
        fin = jnp.where(m_iou >= FG_THRESH, tgt, jnp.zeros((L,), jnp.int32))
        ocls_v[pl.ds(g * L, L)] = fin
        plsc.store_scatter(blob_v, [rows, fc(0)], zero)
        plsc.store_scatter(blob_v, [rows, fc(1)], cx1)
        plsc.store_scatter(blob_v, [rows, fc(2)], cy1)
        plsc.store_scatter(blob_v, [rows, fc(3)], cx2)
        plsc.store_scatter(blob_v, [rows, fc(4)], cy2)
        return carry

    lax.fori_loop(0, ngrp, grp, None, unroll=False)

    pltpu.sync_copy(blob_v, blob_hbm.at[pl.ds(base, b)])
    pltpu.sync_copy(ocls_v, ocls_hbm.at[pl.ds(base, b)])


@jax.jit
def _run(rois, cls_prob, bpred, imb, gtaux_f, gcls_f):
    info = plsc.get_sparse_core_info()
    nc, ns = info.num_cores, info.num_subcores
    nw = nc * ns
    b = -(-N // (nw * L)) * L
    mesh = plsc.VectorSubcoreMesh(core_axis_name="c", subcore_axis_name="s")
    kfn = pl.kernel(
        functools.partial(_body, nc=nc, b=b),
        out_type=[
            jax.ShapeDtypeStruct((N, 5), jnp.float32),
            jax.ShapeDtypeStruct((N,), jnp.int32),
        ],
        mesh=mesh,
        compiler_params=pltpu.CompilerParams(needs_layout_passes=False),
        scratch_types=[
            pltpu.VMEM((b, C), jnp.float32),
            pltpu.VMEM((b, 5), jnp.float32),
            pltpu.VMEM((b, 4 * C), jnp.float32),
            pltpu.VMEM((2 * L,), jnp.float32),
            pltpu.VMEM((5 * G * L,), jnp.float32),
            pltpu.VMEM((G * L,), jnp.int32),
            pltpu.VMEM((b, 5), jnp.float32),
            pltpu.VMEM((b,), jnp.int32),
        ],
    )
    return kfn(rois, cls_prob, bpred, imb, gtaux_f, gcls_f)


def kernel(rois, cls_prob, bbox_pred_tensor, im_info, gt_boxes):
    gt_valid = jnp.cumsum((gt_boxes[:, 2] < 0.01).astype(jnp.int32)) == 0
    qx1 = jnp.where(gt_valid, gt_boxes[:, 0], 2e9)
    qy1 = jnp.where(gt_valid, gt_boxes[:, 1], 2e9)
    qx2 = jnp.where(gt_valid, gt_boxes[:, 2], 0.0)
    qy2 = jnp.where(gt_valid, gt_boxes[:, 3], 0.0)
    qa = (qx2 - qx1 + 1.0) * (qy2 - qy1 + 1.0)
    gtaux = jnp.broadcast_to(
        jnp.stack([qx1, qy1, qx2, qy2, qa])[:, :, None], (5, G, L))
    gcls = jnp.broadcast_to(
        gt_boxes[:, 4].astype(jnp.int32)[:, None], (G, L))
    imb = jnp.concatenate([
        jnp.full((L,), im_info[0, 1] - 1.0, jnp.float32),
        jnp.full((L,), im_info[0, 0] - 1.0, jnp.float32),
    ])
    blob, ocls = _run(rois, cls_prob, bbox_pred_tensor,
                      imb, gtaux.reshape(-1), gcls.reshape(-1))
    return blob, ocls


# speedup vs baseline: 1.9254x; 1.0653x over previous
import functools

import jax
import jax.numpy as jnp
from jax import lax
from jax.experimental import pallas as pl
from jax.experimental.pallas import tpu as pltpu
from jax.experimental.pallas import tpu_sc as plsc

N = 5000
C = 81
G = 64
L = 16
FG_THRESH = 0.5


def _body(rois_hbm, cls_hbm, bpred_hbm, imb_hbm, gtaux_hbm, gcls_hbm,
          blob_hbm, ocls_hbm,
          cls_v, rois_v, bpred_v, imb_v, gtaux_v, gcls_v, am_v,
          ocls_v, sem_cls, sem_in, sem_bp, *, nc, b):
    ngrp = b // L
    wid = lax.axis_index("s") * nc + lax.axis_index("c")
    base = jnp.minimum(wid * b, N - b)

    c_cls = pltpu.async_copy(cls_hbm.at[pl.ds(base, b)], cls_v, sem_cls)
    c_bp = pltpu.async_copy(bpred_hbm.at[pl.ds(base, b)], bpred_v, sem_bp)
    c_rois = pltpu.async_copy(rois_hbm.at[pl.ds(base, b)], rois_v, sem_in)
    c_imb = pltpu.async_copy(imb_hbm, imb_v, sem_in)
    c_gta = pltpu.async_copy(gtaux_hbm, gtaux_v, sem_in)
    c_gcl = pltpu.async_copy(gcls_hbm, gcls_v, sem_in)
    c_cls.wait()

    lanes = jnp.arange(L, dtype=jnp.int32)

    def fc(k):
        return jnp.full((L,), k, jnp.int32)

    def pass1(g, carry):
        rows = g * L + lanes
        m = plsc.load_gather(cls_v, [rows, fc(1)])
        am = jnp.zeros((L,), jnp.int32)
        for cc in range(2, C):
            v = plsc.load_gather(cls_v, [rows, fc(cc)])
            better = v > m
            am = jnp.where(better, cc - 1, am)
            m = jnp.where(better, v, m)
        am_v[pl.ds(g * L, L)] = am * 4
        return carry

    lax.fori_loop(0, ngrp, pass1, None, unroll=2)

    c_rois.wait()
    c_imb.wait()
    c_gta.wait()
    c_gcl.wait()
    c_bp.wait()

    wlim = imb_v[pl.ds(0, L)]
    hlim = imb_v[pl.ds(L, L)]

    def pass2(g, carry):
        rows = g * L + lanes
        am4 = am_v[pl.ds(g * L, L)]
        dx = plsc.load_gather(bpred_v, [rows, am4]) * 0.1
        dy = plsc.load_gather(bpred_v, [rows, am4 + 1]) * 0.1
        dw = plsc.load_gather(bpred_v, [rows, am4 + 2]) * 0.2
        dh = plsc.load_gather(bpred_v, [rows, am4 + 3]) * 0.2
        x1 = plsc.load_gather(rois_v, [rows, fc(1)])
        y1 = plsc.load_gather(rois_v, [rows, fc(2)])
        x2 = plsc.load_gather(rois_v, [rows, fc(3)])
        y2 = plsc.load_gather(rois_v, [rows, fc(4)])
        w = x2 - x1 + 1.0
        h = y2 - y1 + 1.0
        cx = x1 + 0.5 * w
        cy = y1 + 0.5 * h
        pcx = dx * w + cx
        pcy = dy * h + cy
        pw = jnp.exp(dw) * w
        ph = jnp.exp(dh) * h
        bx1 = pcx - 0.5 * pw
        by1 = pcy - 0.5 * ph
        bx2 = pcx + 0.5 * pw
        by2 = pcy + 0.5 * ph
        zero = jnp.zeros((L,), jnp.float32)
        cx1 = jnp.minimum(jnp.maximum(bx1, zero), wlim)
        cy1 = jnp.minimum(jnp.maximum(by1, zero), hlim)
        cx2 = jnp.minimum(jnp.maximum(bx2, zero), wlim)
        cy2 = jnp.minimum(jnp.maximum(by2, zero), hlim)
        cx2p = cx2 + 1.0
        cy2p = cy2 + 1.0
        area_b = (cx2p - cx1) * (cy2p - cy1)

        bi = None
        bu = None
        am2 = jnp.zeros((L,), jnp.int32)
        for g2 in range(G):
            qx1 = gtaux_v[pl.ds((0 * G + g2) * L, L)]
            qy1 = gtaux_v[pl.ds((1 * G + g2) * L, L)]
            qx2p = gtaux_v[pl.ds((2 * G + g2) * L, L)]
            qy2p = gtaux_v[pl.ds((3 * G + g2) * L, L)]
            qa = gtaux_v[pl.ds((4 * G + g2) * L, L)]
            iw = jnp.maximum(
                jnp.minimum(cx2p, qx2p) - jnp.maximum(cx1, qx1), zero)
            ih = jnp.maximum(
                jnp.minimum(cy2p, qy2p) - jnp.maximum(cy1, qy1), zero)
            inter = iw * ih
            union = area_b + qa - inter
            if bi is None:
                bi, bu = inter, union
            else:
                better = inter * bu > bi * union
                am2 = jnp.where(better, g2, am2)
                bi = jnp.where(better, inter, bi)
                bu = jnp.where(better, union, bu)

        m_iou = bi / bu
        tgt = plsc.load_gather(gcls_v, [am2 * L + lanes])
        fin = jnp.where(m_iou >= FG_THRESH, tgt, jnp.zeros((L,), jnp.int32))
        ocls_v[pl.ds(g * L, L)] = fin
        plsc.store_scatter(rois_v, [rows, fc(0)], zero)
        plsc.store_scatter(rois_v, [rows, fc(1)], cx1)
        plsc.store_scatter(rois_v, [rows, fc(2)], cy1)
        plsc.store_scatter(rois_v, [rows, fc(3)], cx2)
        plsc.store_scatter(rois_v, [rows, fc(4)], cy2)
        return carry

    lax.fori_loop(0, ngrp, pass2, None, unroll=2)

    pltpu.sync_copy(rois_v, blob_hbm.at[pl.ds(base, b)])
    pltpu.sync_copy(ocls_v, ocls_hbm.at[pl.ds(base, b)])


@jax.jit
def _run(rois, cls_prob, bpred, imb, gtaux_f, gcls_f):
    info = plsc.get_sparse_core_info()
    nc, ns = info.num_cores, info.num_subcores
    nw = nc * ns
    b = -(-N // (nw * L)) * L
    mesh = plsc.VectorSubcoreMesh(core_axis_name="c", subcore_axis_name="s")
    kfn = pl.kernel(
        functools.partial(_body, nc=nc, b=b),
        out_type=[
            jax.ShapeDtypeStruct((N, 5), jnp.float32),
            jax.ShapeDtypeStruct((N,), jnp.int32),
        ],
        mesh=mesh,
        compiler_params=pltpu.CompilerParams(needs_layout_passes=False),
        scratch_types=[
            pltpu.VMEM((b, C), jnp.float32),
            pltpu.VMEM((b, 5), jnp.float32),
            pltpu.VMEM((b, 4 * C), jnp.float32),
            pltpu.VMEM((2 * L,), jnp.float32),
            pltpu.VMEM((5 * G * L,), jnp.float32),
            pltpu.VMEM((G * L,), jnp.int32),
            pltpu.VMEM((b,), jnp.int32),
            pltpu.VMEM((b,), jnp.int32),
            pltpu.SemaphoreType.DMA,
            pltpu.SemaphoreType.DMA,
            pltpu.SemaphoreType.DMA,
        ],
    )
    return kfn(rois, cls_prob, bpred, imb, gtaux_f, gcls_f)


def kernel(rois, cls_prob, bbox_pred_tensor, im_info, gt_boxes):
    gt_valid = jnp.cumsum((gt_boxes[:, 2] < 0.01).astype(jnp.int32)) == 0
    qx1 = jnp.where(gt_valid, gt_boxes[:, 0], 2e9)
    qy1 = jnp.where(gt_valid, gt_boxes[:, 1], 2e9)
    qx2 = jnp.where(gt_valid, gt_boxes[:, 2], 0.0)
    qy2 = jnp.where(gt_valid, gt_boxes[:, 3], 0.0)
    qa = (qx2 - qx1 + 1.0) * (qy2 - qy1 + 1.0)
    gtaux = jnp.broadcast_to(
        jnp.stack([qx1, qy1, qx2 + 1.0, qy2 + 1.0, qa])[:, :, None],
        (5, G, L))
    gcls = jnp.broadcast_to(
        gt_boxes[:, 4].astype(jnp.int32)[:, None], (G, L))
    imb = jnp.concatenate([
        jnp.full((L,), im_info[0, 1] - 1.0, jnp.float32),
        jnp.full((L,), im_info[0, 0] - 1.0, jnp.float32),
    ])
    blob, ocls = _run(rois, cls_prob, bbox_pred_tensor,
                      imb, gtaux.reshape(-1), gcls.reshape(-1))
    return blob, ocls


# speedup vs baseline: 1.9315x; 1.0032x over previous
import functools

import jax
import jax.numpy as jnp
from jax import lax
from jax.experimental import pallas as pl
from jax.experimental.pallas import tpu as pltpu
from jax.experimental.pallas import tpu_sc as plsc

N = 5000
C = 81
G = 64
L = 16
FG_THRESH = 0.5


def _body(rois_hbm, cls_hbm, bpred_hbm, imb_hbm, gtaux_hbm, gcls_hbm,
          blob_hbm, ocls_hbm,
          cls_v, rois_v, bpred_v, imb_v, gtaux_v, gcls_v, am_v,
          ocls_v, sem_cls, sem_in, sem_bp, *, nc, b):
    ngrp = b // L
    wid = lax.axis_index("s") * nc + lax.axis_index("c")
    base = jnp.minimum(wid * b, N - b)

    c_cls = pltpu.async_copy(cls_hbm.at[pl.ds(base, b)], cls_v, sem_cls)
    c_bp = pltpu.async_copy(bpred_hbm.at[pl.ds(base, b)], bpred_v, sem_bp)
    c_rois = pltpu.async_copy(rois_hbm.at[pl.ds(base, b)], rois_v, sem_in)
    c_imb = pltpu.async_copy(imb_hbm, imb_v, sem_in)
    c_gta = pltpu.async_copy(gtaux_hbm, gtaux_v, sem_in)
    c_gcl = pltpu.async_copy(gcls_hbm, gcls_v, sem_in)
    c_cls.wait()

    lanes = jnp.arange(L, dtype=jnp.int32)

    def fc(k):
        return jnp.full((L,), k, jnp.int32)

    def pass1(g, carry):
        rows = g * L + lanes
        m = plsc.load_gather(cls_v, [rows, fc(1)])
        am = jnp.zeros((L,), jnp.int32)
        for cc in range(2, C):
            v = plsc.load_gather(cls_v, [rows, fc(cc)])
            better = v > m
            am = jnp.where(better, cc - 1, am)
            m = jnp.where(better, v, m)
        am_v[pl.ds(g * L, L)] = am * 4
        return carry

    lax.fori_loop(0, ngrp, pass1, None, unroll=2)

    c_rois.wait()
    c_imb.wait()
    c_gta.wait()
    c_gcl.wait()
    c_bp.wait()

    wlim = imb_v[pl.ds(0, L)]
    hlim = imb_v[pl.ds(L, L)]

    def pass2(g, carry):
        rows = g * L + lanes
        am4 = am_v[pl.ds(g * L, L)]
        dx = plsc.load_gather(bpred_v, [rows, am4]) * 0.1
        dy = plsc.load_gather(bpred_v, [rows, am4 + 1]) * 0.1
        dw = plsc.load_gather(bpred_v, [rows, am4 + 2]) * 0.2
        dh = plsc.load_gather(bpred_v, [rows, am4 + 3]) * 0.2
        x1 = plsc.load_gather(rois_v, [rows, fc(1)])
        y1 = plsc.load_gather(rois_v, [rows, fc(2)])
        x2 = plsc.load_gather(rois_v, [rows, fc(3)])
        y2 = plsc.load_gather(rois_v, [rows, fc(4)])
        w = x2 - x1 + 1.0
        h = y2 - y1 + 1.0
        cx = x1 + 0.5 * w
        cy = y1 + 0.5 * h
        pcx = dx * w + cx
        pcy = dy * h + cy
        pw = jnp.exp(dw) * w
        ph = jnp.exp(dh) * h
        bx1 = pcx - 0.5 * pw
        by1 = pcy - 0.5 * ph
        bx2 = pcx + 0.5 * pw
        by2 = pcy + 0.5 * ph
        zero = jnp.zeros((L,), jnp.float32)
        cx1 = jnp.minimum(jnp.maximum(bx1, zero), wlim)
        cy1 = jnp.minimum(jnp.maximum(by1, zero), hlim)
        cx2 = jnp.minimum(jnp.maximum(bx2, zero), wlim)
        cy2 = jnp.minimum(jnp.maximum(by2, zero), hlim)
        cx2p = cx2 + 1.0
        cy2p = cy2 + 1.0
        area_b = (cx2p - cx1) * (cy2p - cy1)

        bi = None
        bu = None
        am2 = jnp.zeros((L,), jnp.int32)
        for g2 in range(G):
            qx1 = gtaux_v[pl.ds((0 * G + g2) * L, L)]
            qy1 = gtaux_v[pl.ds((1 * G + g2) * L, L)]
            qx2p = gtaux_v[pl.ds((2 * G + g2) * L, L)]
            qy2p = gtaux_v[pl.ds((3 * G + g2) * L, L)]
            qa = gtaux_v[pl.ds((4 * G + g2) * L, L)]
            iw = jnp.maximum(
                jnp.minimum(cx2p, qx2p) - jnp.maximum(cx1, qx1), zero)
            ih = jnp.maximum(
                jnp.minimum(cy2p, qy2p) - jnp.maximum(cy1, qy1), zero)
            inter = iw * ih
            union = area_b + qa - inter
            if bi is None:
                bi, bu = inter, union
            else:
                better = inter * bu > bi * union
                am2 = jnp.where(better, g2, am2)
                bi = jnp.where(better, inter, bi)
                bu = jnp.where(better, union, bu)

        m_iou = bi / bu
        tgt = plsc.load_gather(gcls_v, [am2 * L + lanes])
        fin = jnp.where(m_iou >= FG_THRESH, tgt, jnp.zeros((L,), jnp.int32))
        ocls_v[pl.ds(g * L, L)] = fin
        plsc.store_scatter(rois_v, [rows, fc(0)], zero)
        plsc.store_scatter(rois_v, [rows, fc(1)], cx1)
        plsc.store_scatter(rois_v, [rows, fc(2)], cy1)
        plsc.store_scatter(rois_v, [rows, fc(3)], cx2)
        plsc.store_scatter(rois_v, [rows, fc(4)], cy2)
        return carry

    lax.fori_loop(0, ngrp, pass2, None, unroll=2)

    pltpu.sync_copy(rois_v, blob_hbm.at[pl.ds(base, b)])
    pltpu.sync_copy(ocls_v, ocls_hbm.at[pl.ds(base, b)])


@jax.jit
def _run(rois, cls_prob, bpred, imb, gtaux_f, gcls_f):
    info = plsc.get_sparse_core_info()
    nc, ns = info.num_cores, info.num_subcores
    nw = nc * ns
    b = -(-N // (nw * L)) * L
    mesh = plsc.VectorSubcoreMesh(core_axis_name="c", subcore_axis_name="s")
    kfn = pl.kernel(
        functools.partial(_body, nc=nc, b=b),
        out_type=[
            jax.ShapeDtypeStruct((N, 5), jnp.float32),
            jax.ShapeDtypeStruct((N,), jnp.int32),
        ],
        mesh=mesh,
        compiler_params=pltpu.CompilerParams(
            needs_layout_passes=False, use_tc_tiling_on_sc=True),
        scratch_types=[
            pltpu.VMEM((b, C), jnp.float32),
            pltpu.VMEM((b, 5), jnp.float32),
            pltpu.VMEM((b, 4 * C), jnp.float32),
            pltpu.VMEM((2 * L,), jnp.float32),
            pltpu.VMEM((5 * G * L,), jnp.float32),
            pltpu.VMEM((G * L,), jnp.int32),
            pltpu.VMEM((b,), jnp.int32),
            pltpu.VMEM((b,), jnp.int32),
            pltpu.SemaphoreType.DMA,
            pltpu.SemaphoreType.DMA,
            pltpu.SemaphoreType.DMA,
        ],
    )
    return kfn(rois, cls_prob, bpred, imb, gtaux_f, gcls_f)


def kernel(rois, cls_prob, bbox_pred_tensor, im_info, gt_boxes):
    gt_valid = jnp.cumsum((gt_boxes[:, 2] < 0.01).astype(jnp.int32)) == 0
    qx1 = jnp.where(gt_valid, gt_boxes[:, 0], 2e9)
    qy1 = jnp.where(gt_valid, gt_boxes[:, 1], 2e9)
    qx2 = jnp.where(gt_valid, gt_boxes[:, 2], 0.0)
    qy2 = jnp.where(gt_valid, gt_boxes[:, 3], 0.0)
    qa = (qx2 - qx1 + 1.0) * (qy2 - qy1 + 1.0)
    gtaux = jnp.broadcast_to(
        jnp.stack([qx1, qy1, qx2 + 1.0, qy2 + 1.0, qa])[:, :, None],
        (5, G, L))
    gcls = jnp.broadcast_to(
        gt_boxes[:, 4].astype(jnp.int32)[:, None], (G, L))
    imb = jnp.concatenate([
        jnp.full((L,), im_info[0, 1] - 1.0, jnp.float32),
        jnp.full((L,), im_info[0, 0] - 1.0, jnp.float32),
    ])
    blob, ocls = _run(rois, cls_prob, bbox_pred_tensor,
                      imb, gtaux.reshape(-1), gcls.reshape(-1))
    return blob, ocls


# speedup vs baseline: 2.0037x; 1.0374x over previous
import functools

import jax
import jax.numpy as jnp
from jax import lax
from jax.experimental import pallas as pl
from jax.experimental.pallas import tpu as pltpu
from jax.experimental.pallas import tpu_sc as plsc

N = 5000
C = 81
G = 64
L = 16
BK = 128
NBLK = N // BK
MAIN = NBLK * BK
FG_THRESH = 0.5


def _body(rois_hbm, cls_hbm, bpred_hbm, roist_hbm, clst_hbm, bpredt_hbm,
          imb_hbm, gtaux_hbm, gcls_hbm,
          blobm_hbm, oclsm_hbm, blobt_hbm, oclst_hbm,
          cls_v, rois_v, bpred_v, imb_v, gtaux_v, gcls_v, am_v,
          ocls_v, sem_cls, sem_in, sem_bp, *, nc, nw):
    wid = lax.axis_index("s") * nc + lax.axis_index("c")

    pltpu.sync_copy(imb_hbm, imb_v)
    pltpu.sync_copy(gtaux_hbm, gtaux_v)
    pltpu.sync_copy(gcls_hbm, gcls_v)

    lanes = jnp.arange(L, dtype=jnp.int32)
    wlim = imb_v[pl.ds(0, L)]
    hlim = imb_v[pl.ds(L, L)]

    def pass1(g, carry):
        m = cls_v[1, pl.ds(g * L, L)]
        am = jnp.zeros((L,), jnp.int32)
        for cc in range(2, C):
            v = cls_v[cc, pl.ds(g * L, L)]
            better = v > m
            am = jnp.where(better, cc - 1, am)
            m = jnp.where(better, v, m)
        am_v[pl.ds(g * L, L)] = am * 4
        return carry

    def pass2(g, carry):
        rloc = g * L + lanes
        am4 = am_v[pl.ds(g * L, L)]
        dx = plsc.load_gather(bpred_v, [am4, rloc]) * 0.1
        dy = plsc.load_gather(bpred_v, [am4 + 1, rloc]) * 0.1
        dw = plsc.load_gather(bpred_v, [am4 + 2, rloc]) * 0.2
        dh = plsc.load_gather(bpred_v, [am4 + 3, rloc]) * 0.2
        x1 = rois_v[1, pl.ds(g * L, L)]
        y1 = rois_v[2, pl.ds(g * L, L)]
        x2 = rois_v[3, pl.ds(g * L, L)]
        y2 = rois_v[4, pl.ds(g * L, L)]
        w = x2 - x1 + 1.0
        h = y2 - y1 + 1.0
        cx = x1 + 0.5 * w
        cy = y1 + 0.5 * h
        pcx = dx * w + cx
        pcy = dy * h + cy
        pw = jnp.exp(dw) * w
        ph = jnp.exp(dh) * h
        bx1 = pcx - 0.5 * pw
        by1 = pcy - 0.5 * ph
        bx2 = pcx + 0.5 * pw
        by2 = pcy + 0.5 * ph
        zero = jnp.zeros((L,), jnp.float32)
        cx1 = jnp.minimum(jnp.maximum(bx1, zero), wlim)
        cy1 = jnp.minimum(jnp.maximum(by1, zero), hlim)
        cx2 = jnp.minimum(jnp.maximum(bx2, zero), wlim)
        cy2 = jnp.minimum(jnp.maximum(by2, zero), hlim)
        cx2p = cx2 + 1.0
        cy2p = cy2 + 1.0
        area_b = (cx2p - cx1) * (cy2p - cy1)

        bi = None
        bu = None
        am2 = jnp.zeros((L,), jnp.int32)
        for g2 in range(G):
            qx1 = gtaux_v[pl.ds((0 * G + g2) * L, L)]
            qy1 = gtaux_v[pl.ds((1 * G + g2) * L, L)]
            qx2p = gtaux_v[pl.ds((2 * G + g2) * L, L)]
            qy2p = gtaux_v[pl.ds((3 * G + g2) * L, L)]
            qa = gtaux_v[pl.ds((4 * G + g2) * L, L)]
            iw = jnp.maximum(
                jnp.minimum(cx2p, qx2p) - jnp.maximum(cx1, qx1), zero)
            ih = jnp.maximum(
                jnp.minimum(cy2p, qy2p) - jnp.maximum(cy1, qy1), zero)
            inter = iw * ih
            union = area_b + qa - inter
            if bi is None:
                bi, bu = inter, union
            else:
                better = inter * bu > bi * union
                am2 = jnp.where(better, g2, am2)
                bi = jnp.where(better, inter, bi)
                bu = jnp.where(better, union, bu)

        m_iou = bi / bu
        tgt = plsc.load_gather(gcls_v, [am2 * L + lanes])
        fin = jnp.where(m_iou >= FG_THRESH, tgt, jnp.zeros((L,), jnp.int32))
        ocls_v[pl.ds(g * L, L)] = fin
        rois_v[0, pl.ds(g * L, L)] = zero
        rois_v[1, pl.ds(g * L, L)] = cx1
        rois_v[2, pl.ds(g * L, L)] = cy1
        rois_v[3, pl.ds(g * L, L)] = cx2
        rois_v[4, pl.ds(g * L, L)] = cy2
        return carry

    def do_work(csrc, rsrc, bsrc, sz, blob_dst, ocls_dst):
        c_cls = pltpu.async_copy(csrc, cls_v.at[:, pl.ds(0, sz)], sem_cls)
        c_bp = pltpu.async_copy(bsrc, bpred_v.at[:, pl.ds(0, sz)], sem_bp)
        c_rois = pltpu.async_copy(rsrc, rois_v.at[:, pl.ds(0, sz)], sem_in)
        c_cls.wait()
        lax.fori_loop(0, sz // L, pass1, None, unroll=2)
        c_rois.wait()
        c_bp.wait()
        lax.fori_loop(0, sz // L, pass2, None, unroll=2)
        pltpu.sync_copy(rois_v.at[:, pl.ds(0, sz)], blob_dst)
        pltpu.sync_copy(ocls_v.at[pl.ds(0, sz)], ocls_dst)

    def block(bidx):
        base = pl.multiple_of(bidx * BK, BK)
        do_work(cls_hbm.at[:, pl.ds(base, BK)],
                rois_hbm.at[:, pl.ds(base, BK)],
                bpred_hbm.at[:, pl.ds(base, BK)],
                BK,
                blobm_hbm.at[:, pl.ds(base, BK)],
                oclsm_hbm.at[pl.ds(base, BK)])

    block(wid)

    nextra = NBLK - nw

    @pl.when(wid < nextra)
    def _():
        block(nw + wid)

    @pl.when(wid == nextra)
    def _():
        do_work(clst_hbm.at[:, :], roist_hbm.at[:, :], bpredt_hbm.at[:, :],
                BK, blobt_hbm.at[:, :], oclst_hbm.at[pl.ds(0, BK)])


@jax.jit
def _run(rois_t, cls_t, bpred_t, rois_tl, cls_tl, bpred_tl,
         imb, gtaux_f, gcls_f):
    info = plsc.get_sparse_core_info()
    nc, ns = info.num_cores, info.num_subcores
    nw = nc * ns
    mesh = plsc.VectorSubcoreMesh(core_axis_name="c", subcore_axis_name="s")
    kfn = pl.kernel(
        functools.partial(_body, nc=nc, nw=nw),
        out_type=[
            jax.ShapeDtypeStruct((5, MAIN), jnp.float32),
            jax.ShapeDtypeStruct((MAIN,), jnp.int32),
            jax.ShapeDtypeStruct((5, BK), jnp.float32),
            jax.ShapeDtypeStruct((BK,), jnp.int32),
        ],
        mesh=mesh,
        compiler_params=pltpu.CompilerParams(needs_layout_passes=False),
        scratch_types=[
            pltpu.VMEM((C, BK), jnp.float32),
            pltpu.VMEM((5, BK), jnp.float32),
            pltpu.VMEM((4 * C, BK), jnp.float32),
            pltpu.VMEM((2 * L,), jnp.float32),
            pltpu.VMEM((5 * G * L,), jnp.float32),
            pltpu.VMEM((G * L,), jnp.int32),
            pltpu.VMEM((BK,), jnp.int32),
            pltpu.VMEM((BK,), jnp.int32),
            pltpu.SemaphoreType.DMA,
            pltpu.SemaphoreType.DMA,
            pltpu.SemaphoreType.DMA,
        ],
    )
    return kfn(rois_t, cls_t, bpred_t, rois_tl, cls_tl, bpred_tl,
               imb, gtaux_f, gcls_f)


def kernel(rois, cls_prob, bbox_pred_tensor, im_info, gt_boxes):
    gt_valid = jnp.cumsum((gt_boxes[:, 2] < 0.01).astype(jnp.int32)) == 0
    qx1 = jnp.where(gt_valid, gt_boxes[:, 0], 2e9)
    qy1 = jnp.where(gt_valid, gt_boxes[:, 1], 2e9)
    qx2 = jnp.where(gt_valid, gt_boxes[:, 2], 0.0)
    qy2 = jnp.where(gt_valid, gt_boxes[:, 3], 0.0)
    qa = (qx2 - qx1 + 1.0) * (qy2 - qy1 + 1.0)
    gtaux = jnp.broadcast_to(
        jnp.stack([qx1, qy1, qx2 + 1.0, qy2 + 1.0, qa])[:, :, None],
        (5, G, L))
    gcls = jnp.broadcast_to(
        gt_boxes[:, 4].astype(jnp.int32)[:, None], (G, L))
    imb = jnp.concatenate([
        jnp.full((L,), im_info[0, 1] - 1.0, jnp.float32),
        jnp.full((L,), im_info[0, 0] - 1.0, jnp.float32),
    ])
    blob_m, ocls_m, blob_tl, ocls_tl = _run(
        rois.T, cls_prob.T, bbox_pred_tensor.T,
        rois[N - BK:].T, cls_prob[N - BK:].T, bbox_pred_tensor[N - BK:].T,
        imb, gtaux.reshape(-1), gcls.reshape(-1))
    blob = jnp.concatenate([blob_m, blob_tl[:, BK - (N - MAIN):]], axis=1).T
    ocls = jnp.concatenate([ocls_m, ocls_tl[BK - (N - MAIN):]])
    return blob, ocls


# speedup vs baseline: 2.0083x; 1.0023x over previous
import functools

import jax
import jax.numpy as jnp
from jax import lax
from jax.experimental import pallas as pl
from jax.experimental.pallas import tpu as pltpu
from jax.experimental.pallas import tpu_sc as plsc

N = 5000
C = 81
G = 64
L = 16
BK = 128
NBLK = N // BK
MAIN = NBLK * BK
FG_THRESH = 0.5


def _body(rois_hbm, cls_hbm, bpred_hbm, roist_hbm, clst_hbm, bpredt_hbm,
          imb_hbm, gtaux_hbm, gcls_hbm,
          blobm_hbm, oclsm_hbm, blobt_hbm, oclst_hbm,
          cls_v, rois_v, bpred_v, imb_v, gtaux_v, gcls_v, am_v,
          ocls_v, sem_cls, sem_in, sem_bp, *, nc, nw):
    wid = lax.axis_index("s") * nc + lax.axis_index("c")

    pltpu.sync_copy(imb_hbm, imb_v)
    pltpu.sync_copy(gtaux_hbm, gtaux_v)
    pltpu.sync_copy(gcls_hbm, gcls_v)

    lanes = jnp.arange(L, dtype=jnp.int32)
    wlim = imb_v[pl.ds(0, L)]
    hlim = imb_v[pl.ds(L, L)]

    def pass1(g, carry):
        m = cls_v[1, pl.ds(g * L, L)]
        am = jnp.zeros((L,), jnp.int32)
        for cc in range(2, C):
            v = cls_v[cc, pl.ds(g * L, L)]
            better = v > m
            am = jnp.where(better, cc - 1, am)
            m = jnp.where(better, v, m)
        am_v[pl.ds(g * L, L)] = am * 4
        return carry

    def pass2(g, carry):
        rloc = g * L + lanes
        am4 = am_v[pl.ds(g * L, L)]
        dx = plsc.load_gather(bpred_v, [am4, rloc]) * 0.1
        dy = plsc.load_gather(bpred_v, [am4 + 1, rloc]) * 0.1
        dw = plsc.load_gather(bpred_v, [am4 + 2, rloc]) * 0.2
        dh = plsc.load_gather(bpred_v, [am4 + 3, rloc]) * 0.2
        x1 = rois_v[1, pl.ds(g * L, L)]
        y1 = rois_v[2, pl.ds(g * L, L)]
        x2 = rois_v[3, pl.ds(g * L, L)]
        y2 = rois_v[4, pl.ds(g * L, L)]
        w = x2 - x1 + 1.0
        h = y2 - y1 + 1.0
        cx = x1 + 0.5 * w
        cy = y1 + 0.5 * h
        pcx = dx * w + cx
        pcy = dy * h + cy
        pw = jnp.exp(dw) * w
        ph = jnp.exp(dh) * h
        bx1 = pcx - 0.5 * pw
        by1 = pcy - 0.5 * ph
        bx2 = pcx + 0.5 * pw
        by2 = pcy + 0.5 * ph
        zero = jnp.zeros((L,), jnp.float32)
        cx1 = jnp.minimum(jnp.maximum(bx1, zero), wlim)
        cy1 = jnp.minimum(jnp.maximum(by1, zero), hlim)
        cx2 = jnp.minimum(jnp.maximum(bx2, zero), wlim)
        cy2 = jnp.minimum(jnp.maximum(by2, zero), hlim)
        cx2p = cx2 + 1.0
        cy2p = cy2 + 1.0
        area_b = (cx2p - cx1) * (cy2p - cy1)

        bi = None
        bu = None
        am2 = jnp.zeros((L,), jnp.int32)
        for g2 in range(G):
            qx1 = gtaux_v[pl.ds((0 * G + g2) * L, L)]
            qy1 = gtaux_v[pl.ds((1 * G + g2) * L, L)]
            qx2p = gtaux_v[pl.ds((2 * G + g2) * L, L)]
            qy2p = gtaux_v[pl.ds((3 * G + g2) * L, L)]
            qa = gtaux_v[pl.ds((4 * G + g2) * L, L)]
            iw = jnp.maximum(
                jnp.minimum(cx2p, qx2p) - jnp.maximum(cx1, qx1), zero)
            ih = jnp.maximum(
                jnp.minimum(cy2p, qy2p) - jnp.maximum(cy1, qy1), zero)
            inter = iw * ih
            union = area_b + qa - inter
            if bi is None:
                bi, bu = inter, union
            else:
                better = inter * bu > bi * union
                am2 = jnp.where(better, g2, am2)
                bi = jnp.where(better, inter, bi)
                bu = jnp.where(better, union, bu)

        m_iou = bi / bu
        tgt = plsc.load_gather(gcls_v, [am2 * L + lanes])
        fin = jnp.where(m_iou >= FG_THRESH, tgt, jnp.zeros((L,), jnp.int32))
        ocls_v[pl.ds(g * L, L)] = fin
        rois_v[0, pl.ds(g * L, L)] = zero
        rois_v[1, pl.ds(g * L, L)] = cx1
        rois_v[2, pl.ds(g * L, L)] = cy1
        rois_v[3, pl.ds(g * L, L)] = cx2
        rois_v[4, pl.ds(g * L, L)] = cy2
        return carry

    def do_work(csrc, rsrc, bsrc, sz, blob_dst, ocls_dst):
        c_cls = pltpu.async_copy(csrc, cls_v.at[:, pl.ds(0, sz)], sem_cls)
        c_bp = pltpu.async_copy(bsrc, bpred_v.at[:, pl.ds(0, sz)], sem_bp)
        c_rois = pltpu.async_copy(rsrc, rois_v.at[:, pl.ds(0, sz)], sem_in)
        c_cls.wait()
        lax.fori_loop(0, sz // L, pass1, None, unroll=False)
        c_rois.wait()
        c_bp.wait()
        lax.fori_loop(0, sz // L, pass2, None, unroll=False)
        pltpu.sync_copy(rois_v.at[:, pl.ds(0, sz)], blob_dst)
        pltpu.sync_copy(ocls_v.at[pl.ds(0, sz)], ocls_dst)

    nextra = NBLK - nw

    def block(i, carry):
        base = pl.multiple_of((wid + i * nw) * BK, BK)
        do_work(cls_hbm.at[:, pl.ds(base, BK)],
                rois_hbm.at[:, pl.ds(base, BK)],
                bpred_hbm.at[:, pl.ds(base, BK)],
                BK,
                blobm_hbm.at[:, pl.ds(base, BK)],
                oclsm_hbm.at[pl.ds(base, BK)])
        return carry

    nblk_w = 1 + (wid < nextra).astype(jnp.int32)
    lax.fori_loop(0, nblk_w, block, None, unroll=False)

    @pl.when(wid == nextra)
    def _():
        do_work(clst_hbm.at[:, :], roist_hbm.at[:, :], bpredt_hbm.at[:, :],
                BK, blobt_hbm.at[:, :], oclst_hbm.at[pl.ds(0, BK)])


@jax.jit
def _run(rois_t, cls_t, bpred_t, rois_tl, cls_tl, bpred_tl,
         imb, gtaux_f, gcls_f):
    info = plsc.get_sparse_core_info()
    nc, ns = info.num_cores, info.num_subcores
    nw = nc * ns
    mesh = plsc.VectorSubcoreMesh(core_axis_name="c", subcore_axis_name="s")
    kfn = pl.kernel(
        functools.partial(_body, nc=nc, nw=nw),
        out_type=[
            jax.ShapeDtypeStruct((5, MAIN), jnp.float32),
            jax.ShapeDtypeStruct((MAIN,), jnp.int32),
            jax.ShapeDtypeStruct((5, BK), jnp.float32),
            jax.ShapeDtypeStruct((BK,), jnp.int32),
        ],
        mesh=mesh,
        compiler_params=pltpu.CompilerParams(needs_layout_passes=False),
        scratch_types=[
            pltpu.VMEM((C, BK), jnp.float32),
            pltpu.VMEM((5, BK), jnp.float32),
            pltpu.VMEM((4 * C, BK), jnp.float32),
            pltpu.VMEM((2 * L,), jnp.float32),
            pltpu.VMEM((5 * G * L,), jnp.float32),
            pltpu.VMEM((G * L,), jnp.int32),
            pltpu.VMEM((BK,), jnp.int32),
            pltpu.VMEM((BK,), jnp.int32),
            pltpu.SemaphoreType.DMA,
            pltpu.SemaphoreType.DMA,
            pltpu.SemaphoreType.DMA,
        ],
    )
    return kfn(rois_t, cls_t, bpred_t, rois_tl, cls_tl, bpred_tl,
               imb, gtaux_f, gcls_f)


def kernel(rois, cls_prob, bbox_pred_tensor, im_info, gt_boxes):
    gt_valid = jnp.cumsum((gt_boxes[:, 2] < 0.01).astype(jnp.int32)) == 0
    qx1 = jnp.where(gt_valid, gt_boxes[:, 0], 2e9)
    qy1 = jnp.where(gt_valid, gt_boxes[:, 1], 2e9)
    qx2 = jnp.where(gt_valid, gt_boxes[:, 2], 0.0)
    qy2 = jnp.where(gt_valid, gt_boxes[:, 3], 0.0)
    qa = (qx2 - qx1 + 1.0) * (qy2 - qy1 + 1.0)
    gtaux = jnp.broadcast_to(
        jnp.stack([qx1, qy1, qx2 + 1.0, qy2 + 1.0, qa])[:, :, None],
        (5, G, L))
    gcls = jnp.broadcast_to(
        gt_boxes[:, 4].astype(jnp.int32)[:, None], (G, L))
    imb = jnp.concatenate([
        jnp.full((L,), im_info[0, 1] - 1.0, jnp.float32),
        jnp.full((L,), im_info[0, 0] - 1.0, jnp.float32),
    ])
    blob_m, ocls_m, blob_tl, ocls_tl = _run(
        rois.T, cls_prob.T, bbox_pred_tensor.T,
        rois[N - BK:].T, cls_prob[N - BK:].T, bbox_pred_tensor[N - BK:].T,
        imb, gtaux.reshape(-1), gcls.reshape(-1))
    blob = jnp.concatenate([blob_m, blob_tl[:, BK - (N - MAIN):]], axis=1).T
    ocls = jnp.concatenate([ocls_m, ocls_tl[BK - (N - MAIN):]])
    return blob, ocls


# speedup vs baseline: 2.0884x; 1.0399x over previous
import functools

import jax
import jax.numpy as jnp
from jax import lax
from jax.experimental import pallas as pl
from jax.experimental.pallas import tpu as pltpu
from jax.experimental.pallas import tpu_sc as plsc

N = 5000
C = 81
G = 64
L = 16
BK = 128
NBLK = N // BK
MAIN = NBLK * BK
FG_THRESH = 0.5


def _body(rois_hbm, cls_hbm, bpred_hbm, roist_hbm, clst_hbm, bpredt_hbm,
          imb_hbm, gtaux_hbm, gcls_hbm,
          blobm_hbm, oclsm_hbm, blobt_hbm, oclst_hbm,
          cls_v, rois_v, bpred_v, imb_v, gtaux_v, gcls_v, am_v,
          ocls_v, sem_cls, sem_in, sem_bp, *, nc, nw):
    wid = lax.axis_index("s") * nc + lax.axis_index("c")

    pltpu.sync_copy(imb_hbm, imb_v)
    pltpu.sync_copy(gtaux_hbm, gtaux_v)
    pltpu.sync_copy(gcls_hbm, gcls_v)

    lanes = jnp.arange(L, dtype=jnp.int32)
    wlim = imb_v[pl.ds(0, L)]
    hlim = imb_v[pl.ds(L, L)]

    def pass1(g, carry):
        m = cls_v[1, pl.ds(g * L, L)]
        am = jnp.zeros((L,), jnp.int32)
        for cc in range(2, C):
            v = cls_v[cc, pl.ds(g * L, L)]
            better = v > m
            am = jnp.where(better, cc - 1, am)
            m = jnp.where(better, v, m)
        am_v[pl.ds(g * L, L)] = am * 4
        return carry

    def pass2(g, carry):
        rloc = g * L + lanes
        am4 = am_v[pl.ds(g * L, L)]
        dx = plsc.load_gather(bpred_v, [am4, rloc]) * 0.1
        dy = plsc.load_gather(bpred_v, [am4 + 1, rloc]) * 0.1
        dw = plsc.load_gather(bpred_v, [am4 + 2, rloc]) * 0.2
        dh = plsc.load_gather(bpred_v, [am4 + 3, rloc]) * 0.2
        x1 = rois_v[1, pl.ds(g * L, L)]
        y1 = rois_v[2, pl.ds(g * L, L)]
        x2 = rois_v[3, pl.ds(g * L, L)]
        y2 = rois_v[4, pl.ds(g * L, L)]
        w = x2 - x1 + 1.0
        h = y2 - y1 + 1.0
        cx = x1 + 0.5 * w
        cy = y1 + 0.5 * h
        pcx = dx * w + cx
        pcy = dy * h + cy
        pw = jnp.exp(dw) * w
        ph = jnp.exp(dh) * h
        bx1 = pcx - 0.5 * pw
        by1 = pcy - 0.5 * ph
        bx2 = pcx + 0.5 * pw
        by2 = pcy + 0.5 * ph
        zero = jnp.zeros((L,), jnp.float32)
        cx1 = jnp.minimum(jnp.maximum(bx1, zero), wlim)
        cy1 = jnp.minimum(jnp.maximum(by1, zero), hlim)
        cx2 = jnp.minimum(jnp.maximum(bx2, zero), wlim)
        cy2 = jnp.minimum(jnp.maximum(by2, zero), hlim)
        cx2p = cx2 + 1.0
        cy2p = cy2 + 1.0
        area_b = (cx2p - cx1) * (cy2p - cy1)

        bi = None
        bu = None
        am2 = jnp.zeros((L,), jnp.int32)
        for g2 in range(G):
            qx1 = gtaux_v[pl.ds((0 * G + g2) * L, L)]
            qy1 = gtaux_v[pl.ds((1 * G + g2) * L, L)]
            qx2p = gtaux_v[pl.ds((2 * G + g2) * L, L)]
            qy2p = gtaux_v[pl.ds((3 * G + g2) * L, L)]
            qa = gtaux_v[pl.ds((4 * G + g2) * L, L)]
            iw = jnp.maximum(
                jnp.minimum(cx2p, qx2p) - jnp.maximum(cx1, qx1), zero)
            ih = jnp.maximum(
                jnp.minimum(cy2p, qy2p) - jnp.maximum(cy1, qy1), zero)
            inter = iw * ih
            union = area_b + qa - inter
            if bi is None:
                bi, bu = inter, union
            else:
                better = inter * bu > bi * union
                am2 = jnp.where(better, g2, am2)
                bi = jnp.where(better, inter, bi)
                bu = jnp.where(better, union, bu)

        m_iou = bi / bu
        tgt = plsc.load_gather(gcls_v, [am2 * L + lanes])
        fin = jnp.where(m_iou >= FG_THRESH, tgt, jnp.zeros((L,), jnp.int32))
        ocls_v[pl.ds(g * L, L)] = fin
        rois_v[0, pl.ds(g * L, L)] = zero
        rois_v[1, pl.ds(g * L, L)] = cx1
        rois_v[2, pl.ds(g * L, L)] = cy1
        rois_v[3, pl.ds(g * L, L)] = cx2
        rois_v[4, pl.ds(g * L, L)] = cy2
        return carry

    def do_work(csrc, rsrc, bsrc, sz, blob_dst, ocls_dst):
        c_cls = pltpu.async_copy(csrc, cls_v.at[:, pl.ds(0, sz)], sem_cls)
        c_bp = pltpu.async_copy(bsrc, bpred_v.at[:, pl.ds(0, sz)], sem_bp)
        c_rois = pltpu.async_copy(rsrc, rois_v.at[:, pl.ds(0, sz)], sem_in)
        c_cls.wait()
        lax.fori_loop(0, sz // L, pass1, None, unroll=False)
        c_rois.wait()
        c_bp.wait()
        lax.fori_loop(0, sz // L, pass2, None, unroll=False)
        pltpu.sync_copy(rois_v.at[:, pl.ds(0, sz)], blob_dst)
        pltpu.sync_copy(ocls_v.at[pl.ds(0, sz)], ocls_dst)

    nextra = NBLK - nw

    def block(i, carry):
        base = pl.multiple_of((wid + i * nw) * BK, BK)
        do_work(cls_hbm.at[:, pl.ds(base, BK)],
                rois_hbm.at[:, pl.ds(base, BK)],
                bpred_hbm.at[:, pl.ds(base, BK)],
                BK,
                blobm_hbm.at[:, pl.ds(base, BK)],
                oclsm_hbm.at[pl.ds(base, BK)])
        return carry

    nblk_w = 1 + (wid < nextra).astype(jnp.int32)
    lax.fori_loop(0, nblk_w, block, None, unroll=False)

    @pl.when(wid == nextra)
    def _():
        do_work(clst_hbm.at[:, :], roist_hbm.at[:, :], bpredt_hbm.at[:, :],
                BK, blobt_hbm.at[:, :], oclst_hbm.at[pl.ds(0, BK)])


@jax.jit
def _run(rois_t, cls_t, bpred_t, rois_tl, cls_tl, bpred_tl,
         imb, gtaux_f, gcls_f):
    info = plsc.get_sparse_core_info()
    nc, ns = info.num_cores, info.num_subcores
    nw = nc * ns
    mesh = plsc.VectorSubcoreMesh(core_axis_name="c", subcore_axis_name="s")
    kfn = pl.kernel(
        functools.partial(_body, nc=nc, nw=nw),
        out_type=[
            jax.ShapeDtypeStruct((5, MAIN), jnp.float32),
            jax.ShapeDtypeStruct((MAIN,), jnp.int32),
            jax.ShapeDtypeStruct((5, BK), jnp.float32),
            jax.ShapeDtypeStruct((BK,), jnp.int32),
        ],
        mesh=mesh,
        compiler_params=pltpu.CompilerParams(needs_layout_passes=False),
        scratch_types=[
            pltpu.VMEM((C, BK), jnp.float32),
            pltpu.VMEM((5, BK), jnp.float32),
            pltpu.VMEM((4 * C, BK), jnp.float32),
            pltpu.VMEM((2 * L,), jnp.float32),
            pltpu.VMEM((5 * G * L,), jnp.float32),
            pltpu.VMEM((G * L,), jnp.int32),
            pltpu.VMEM((BK,), jnp.int32),
            pltpu.VMEM((BK,), jnp.int32),
            pltpu.SemaphoreType.DMA,
            pltpu.SemaphoreType.DMA,
            pltpu.SemaphoreType.DMA,
        ],
    )
    return kfn(rois_t, cls_t, bpred_t, rois_tl, cls_tl, bpred_tl,
               imb, gtaux_f, gcls_f)


def kernel(rois, cls_prob, bbox_pred_tensor, im_info, gt_boxes):
    gt_valid = jnp.cumsum((gt_boxes[:, 2] < 0.01).astype(jnp.int32)) == 0
    qx1 = jnp.where(gt_valid, gt_boxes[:, 0], 2e9)
    qy1 = jnp.where(gt_valid, gt_boxes[:, 1], 2e9)
    qx2 = jnp.where(gt_valid, gt_boxes[:, 2], 0.0)
    qy2 = jnp.where(gt_valid, gt_boxes[:, 3], 0.0)
    qa = (qx2 - qx1 + 1.0) * (qy2 - qy1 + 1.0)
    gtaux = jnp.broadcast_to(
        jnp.stack([qx1, qy1, qx2 + 1.0, qy2 + 1.0, qa])[:, :, None],
        (5, G, L))
    gcls = jnp.broadcast_to(
        gt_boxes[:, 4].astype(jnp.int32)[:, None], (G, L))
    imb = jnp.concatenate([
        jnp.full((L,), im_info[0, 1] - 1.0, jnp.float32),
        jnp.full((L,), im_info[0, 0] - 1.0, jnp.float32),
    ])
    pad = ((0, 0), (0, BK - (N - MAIN)))
    rois_tl = jnp.pad(rois[MAIN:].T, pad)
    cls_tl = jnp.pad(cls_prob[MAIN:].T, pad)
    bpred_tl = jnp.pad(bbox_pred_tensor[MAIN:].T, pad)
    blob_m, ocls_m, blob_tl, ocls_tl = _run(
        rois.T, cls_prob.T, bbox_pred_tensor.T,
        rois_tl, cls_tl, bpred_tl,
        imb, gtaux.reshape(-1), gcls.reshape(-1))
    blob = jnp.concatenate([blob_m, blob_tl[:, :N - MAIN]], axis=1).T
    ocls = jnp.concatenate([ocls_m, ocls_tl[:N - MAIN]])
    return blob, ocls


# speedup vs baseline: 2.3432x; 1.1220x over previous
import functools

import jax
import jax.numpy as jnp
from jax import lax
from jax.experimental import pallas as pl
from jax.experimental.pallas import tpu as pltpu
from jax.experimental.pallas import tpu_sc as plsc

N = 5000
C = 81
G = 64
L = 16
BK = 128
NBLK = N // BK
MAIN = NBLK * BK
FG_THRESH = 0.5


def _body(rois_hbm, cls_hbm, bpred_hbm, roist_hbm, clst_hbm, bpredt_hbm,
          imb_hbm, gtaux_hbm, gcls_hbm,
          blobm_hbm, oclsm_hbm, blobt_hbm, oclst_hbm,
          cls_v, rois_v, bpred_v, cls2_v, rois2_v, bpred2_v,
          imb_v, gtaux_v, gcls_v, am_v, ocls_v, ocls2_v,
          sa_cls, sa_in, sa_bp, sb_cls, sb_in, sb_bp, *, nc, nw):
    wid = lax.axis_index("s") * nc + lax.axis_index("c")
    nextra = NBLK - nw
    base1 = pl.multiple_of(wid * BK, BK)
    second_main = wid < nextra
    second_tail = wid == nextra

    a_cls = pltpu.async_copy(cls_hbm.at[:, pl.ds(base1, BK)], cls_v, sa_cls)
    a_bp = pltpu.async_copy(bpred_hbm.at[:, pl.ds(base1, BK)], bpred_v, sa_bp)
    a_rois = pltpu.async_copy(rois_hbm.at[:, pl.ds(base1, BK)], rois_v, sa_in)

    @pl.when(second_main)
    def _():
        base2 = pl.multiple_of((wid + nw) * BK, BK)
        pltpu.async_copy(cls_hbm.at[:, pl.ds(base2, BK)], cls2_v, sb_cls)
        pltpu.async_copy(bpred_hbm.at[:, pl.ds(base2, BK)], bpred2_v, sb_bp)
        pltpu.async_copy(rois_hbm.at[:, pl.ds(base2, BK)], rois2_v, sb_in)

    @pl.when(second_tail)
    def _():
        pltpu.async_copy(clst_hbm.at[:, :], cls2_v, sb_cls)
        pltpu.async_copy(bpredt_hbm.at[:, :], bpred2_v, sb_bp)
        pltpu.async_copy(roist_hbm.at[:, :], rois2_v, sb_in)

    pltpu.sync_copy(imb_hbm, imb_v)
    pltpu.sync_copy(gtaux_hbm, gtaux_v)
    pltpu.sync_copy(gcls_hbm, gcls_v)

    lanes = jnp.arange(L, dtype=jnp.int32)
    wlim = imb_v[pl.ds(0, L)]
    hlim = imb_v[pl.ds(L, L)]

    def make_pass1(cv):
        def pass1(g, carry):
            m = cv[1, pl.ds(g * L, L)]
            am = jnp.zeros((L,), jnp.int32)
            for cc in range(2, C):
                v = cv[cc, pl.ds(g * L, L)]
                better = v > m
                am = jnp.where(better, cc - 1, am)
                m = jnp.where(better, v, m)
            am_v[pl.ds(g * L, L)] = am * 4
            return carry
        return pass1

    def make_pass2(bv, rv, ov):
        def pass2(g, carry):
            rloc = g * L + lanes
            am4 = am_v[pl.ds(g * L, L)]
            dx = plsc.load_gather(bv, [am4, rloc]) * 0.1
            dy = plsc.load_gather(bv, [am4 + 1, rloc]) * 0.1
            dw = plsc.load_gather(bv, [am4 + 2, rloc]) * 0.2
            dh = plsc.load_gather(bv, [am4 + 3, rloc]) * 0.2
            x1 = rv[1, pl.ds(g * L, L)]
            y1 = rv[2, pl.ds(g * L, L)]
            x2 = rv[3, pl.ds(g * L, L)]
            y2 = rv[4, pl.ds(g * L, L)]
            w = x2 - x1 + 1.0
            h = y2 - y1 + 1.0
            cx = x1 + 0.5 * w
            cy = y1 + 0.5 * h
            pcx = dx * w + cx
            pcy = dy * h + cy
            pw = jnp.exp(dw) * w
            ph = jnp.exp(dh) * h
            bx1 = pcx - 0.5 * pw
            by1 = pcy - 0.5 * ph
            bx2 = pcx + 0.5 * pw
            by2 = pcy + 0.5 * ph
            zero = jnp.zeros((L,), jnp.float32)
            cx1 = jnp.minimum(jnp.maximum(bx1, zero), wlim)
            cy1 = jnp.minimum(jnp.maximum(by1, zero), hlim)
            cx2 = jnp.minimum(jnp.maximum(bx2, zero), wlim)
            cy2 = jnp.minimum(jnp.maximum(by2, zero), hlim)
            cx2p = cx2 + 1.0
            cy2p = cy2 + 1.0
            area_b = (cx2p - cx1) * (cy2p - cy1)

            bi = None
            bu = None
            am2 = jnp.zeros((L,), jnp.int32)
            for g2 in range(G):
                qx1 = gtaux_v[pl.ds((0 * G + g2) * L, L)]
                qy1 = gtaux_v[pl.ds((1 * G + g2) * L, L)]
                qx2p = gtaux_v[pl.ds((2 * G + g2) * L, L)]
                qy2p = gtaux_v[pl.ds((3 * G + g2) * L, L)]
                qa = gtaux_v[pl.ds((4 * G + g2) * L, L)]
                iw = jnp.maximum(
                    jnp.minimum(cx2p, qx2p) - jnp.maximum(cx1, qx1), zero)
                ih = jnp.maximum(
                    jnp.minimum(cy2p, qy2p) - jnp.maximum(cy1, qy1), zero)
                inter = iw * ih
                union = area_b + qa - inter
                if bi is None:
                    bi, bu = inter, union
                else:
                    better = inter * bu > bi * union
                    am2 = jnp.where(better, g2, am2)
                    bi = jnp.where(better, inter, bi)
                    bu = jnp.where(better, union, bu)

            m_iou = bi / bu
            tgt = plsc.load_gather(gcls_v, [am2 * L + lanes])
            fin = jnp.where(m_iou >= FG_THRESH, tgt,
                            jnp.zeros((L,), jnp.int32))
            ov[pl.ds(g * L, L)] = fin
            rv[0, pl.ds(g * L, L)] = zero
            rv[1, pl.ds(g * L, L)] = cx1
            rv[2, pl.ds(g * L, L)] = cy1
            rv[3, pl.ds(g * L, L)] = cx2
            rv[4, pl.ds(g * L, L)] = cy2
            return carry
        return pass2

    ngrp = BK // L

    a_cls.wait()
    lax.fori_loop(0, ngrp, make_pass1(cls_v), None, unroll=False)
    a_rois.wait()
    a_bp.wait()
    lax.fori_loop(0, ngrp, make_pass2(bpred_v, rois_v, ocls_v), None,
                  unroll=False)
    pltpu.sync_copy(rois_v, blobm_hbm.at[:, pl.ds(base1, BK)])
    pltpu.sync_copy(ocls_v, oclsm_hbm.at[pl.ds(base1, BK)])

    @pl.when(second_main | second_tail)
    def _():
        pltpu.make_async_copy(cls_hbm.at[:, pl.ds(0, BK)], cls2_v,
                              sb_cls).wait()
        lax.fori_loop(0, ngrp, make_pass1(cls2_v), None, unroll=False)
        pltpu.make_async_copy(rois_hbm.at[:, pl.ds(0, BK)], rois2_v,
                              sb_in).wait()
        pltpu.make_async_copy(bpred_hbm.at[:, pl.ds(0, BK)], bpred2_v,
                              sb_bp).wait()
        lax.fori_loop(0, ngrp, make_pass2(bpred2_v, rois2_v, ocls2_v),
                      None, unroll=False)

        @pl.when(second_main)
        def _():
            base2 = pl.multiple_of((wid + nw) * BK, BK)
            pltpu.sync_copy(rois2_v, blobm_hbm.at[:, pl.ds(base2, BK)])
            pltpu.sync_copy(ocls2_v, oclsm_hbm.at[pl.ds(base2, BK)])

        @pl.when(second_tail)
        def _():
            pltpu.sync_copy(rois2_v, blobt_hbm.at[:, :])
            pltpu.sync_copy(ocls2_v, oclst_hbm.at[pl.ds(0, BK)])


@jax.jit
def _run(rois_t, cls_t, bpred_t, rois_tl, cls_tl, bpred_tl,
         imb, gtaux_f, gcls_f):
    info = plsc.get_sparse_core_info()
    nc, ns = info.num_cores, info.num_subcores
    nw = nc * ns
    mesh = plsc.VectorSubcoreMesh(core_axis_name="c", subcore_axis_name="s")
    kfn = pl.kernel(
        functools.partial(_body, nc=nc, nw=nw),
        out_type=[
            jax.ShapeDtypeStruct((5, MAIN), jnp.float32),
            jax.ShapeDtypeStruct((MAIN,), jnp.int32),
            jax.ShapeDtypeStruct((5, BK), jnp.float32),
            jax.ShapeDtypeStruct((BK,), jnp.int32),
        ],
        mesh=mesh,
        compiler_params=pltpu.CompilerParams(needs_layout_passes=False),
        scratch_types=[
            pltpu.VMEM((C, BK), jnp.float32),
            pltpu.VMEM((5, BK), jnp.float32),
            pltpu.VMEM((4 * C, BK), jnp.float32),
            pltpu.VMEM((C, BK), jnp.float32),
            pltpu.VMEM((5, BK), jnp.float32),
            pltpu.VMEM((4 * C, BK), jnp.float32),
            pltpu.VMEM((2 * L,), jnp.float32),
            pltpu.VMEM((5 * G * L,), jnp.float32),
            pltpu.VMEM((G * L,), jnp.int32),
            pltpu.VMEM((BK,), jnp.int32),
            pltpu.VMEM((BK,), jnp.int32),
            pltpu.VMEM((BK,), jnp.int32),
            pltpu.SemaphoreType.DMA,
            pltpu.SemaphoreType.DMA,
            pltpu.SemaphoreType.DMA,
            pltpu.SemaphoreType.DMA,
            pltpu.SemaphoreType.DMA,
            pltpu.SemaphoreType.DMA,
        ],
    )
    return kfn(rois_t, cls_t, bpred_t, rois_tl, cls_tl, bpred_tl,
               imb, gtaux_f, gcls_f)


def kernel(rois, cls_prob, bbox_pred_tensor, im_info, gt_boxes):
    gt_valid = jnp.cumsum((gt_boxes[:, 2] < 0.01).astype(jnp.int32)) == 0
    qx1 = jnp.where(gt_valid, gt_boxes[:, 0], 2e9)
    qy1 = jnp.where(gt_valid, gt_boxes[:, 1], 2e9)
    qx2 = jnp.where(gt_valid, gt_boxes[:, 2], 0.0)
    qy2 = jnp.where(gt_valid, gt_boxes[:, 3], 0.0)
    qa = (qx2 - qx1 + 1.0) * (qy2 - qy1 + 1.0)
    gtaux = jnp.broadcast_to(
        jnp.stack([qx1, qy1, qx2 + 1.0, qy2 + 1.0, qa])[:, :, None],
        (5, G, L))
    gcls = jnp.broadcast_to(
        gt_boxes[:, 4].astype(jnp.int32)[:, None], (G, L))
    imb = jnp.concatenate([
        jnp.full((L,), im_info[0, 1] - 1.0, jnp.float32),
        jnp.full((L,), im_info[0, 0] - 1.0, jnp.float32),
    ])
    pad = ((0, 0), (0, BK - (N - MAIN)))
    rois_tl = jnp.pad(rois[MAIN:].T, pad)
    cls_tl = jnp.pad(cls_prob[MAIN:].T, pad)
    bpred_tl = jnp.pad(bbox_pred_tensor[MAIN:].T, pad)
    blob_m, ocls_m, blob_tl, ocls_tl = _run(
        rois.T, cls_prob.T, bbox_pred_tensor.T,
        rois_tl, cls_tl, bpred_tl,
        imb, gtaux.reshape(-1), gcls.reshape(-1))
    blob = jnp.concatenate([blob_m, blob_tl[:, :N - MAIN]], axis=1).T
    ocls = jnp.concatenate([ocls_m, ocls_tl[:N - MAIN]])
    return blob, ocls
